# baseline (speedup 1.0000x reference)
import functools

import jax
import jax.numpy as jnp
from jax import lax
from jax.experimental import pallas as pl
from jax.experimental.pallas import tpu as pltpu
from jax.experimental.pallas import tpu_sc as plsc

NUM_ENTITIES = 1000000
NUM_RELATIONS = 1000
EMBED_DIM = 64
BATCH = 16384

NC = 2
NS = 16
NW = NC * NS
BPW = BATCH // NW
LANES = 16
WIDE = 2 * EMBED_DIM
CH = 128
NCH = BPW // CH
VPR = EMBED_DIM // LANES


def _make_sc_lookup():
  mesh = plsc.VectorSubcoreMesh(core_axis_name="c", subcore_axis_name="s")

  @functools.partial(
      pl.kernel,
      mesh=mesh,
      compiler_params=pltpu.CompilerParams(needs_layout_passes=False),
      out_type=(
          jax.ShapeDtypeStruct((BATCH, EMBED_DIM), jnp.float32),
          jax.ShapeDtypeStruct((BATCH, EMBED_DIM), jnp.float32),
      ),
      scratch_types=[
          pltpu.VMEM((BPW,), jnp.int32),
          pltpu.VMEM((BPW,), jnp.int32),
          pltpu.VMEM((NCH, CH), jnp.int32),
          pltpu.VMEM((NCH, CH), jnp.int32),
          pltpu.VMEM((CH, WIDE), jnp.float32),
          pltpu.VMEM((CH, WIDE), jnp.float32),
          pltpu.VMEM((CH, EMBED_DIM), jnp.float32),
          pltpu.SemaphoreType.DMA,
          pltpu.SemaphoreType.DMA,
      ],
  )
  def lookup(heads_hbm, rels_hbm, ent_hbm, rel_hbm, out_h, out_r,
             hidx_v, ridx_v, hsup, rsup, blk_a, blk_b, row, sem_a, sem_b):
    wid = lax.axis_index("s") * NC + lax.axis_index("c")
    base = wid * BPW
    pltpu.sync_copy(heads_hbm.at[wid], hidx_v)
    pltpu.sync_copy(rels_hbm.at[wid], ridx_v)

    def sup_ids(idx_v, sup):
      def body(g, _):
        gb = g * LANES
        vec = idx_v[pl.ds(gb, LANES)] >> 1
        sup[lax.div(g, CH // LANES),
            pl.ds(lax.rem(g, CH // LANES) * LANES, LANES)] = vec
        return _
      lax.fori_loop(0, BPW // LANES, body, 0)

    sup_ids(hidx_v, hsup)
    sup_ids(ridx_v, rsup)

    blks = (blk_a, blk_b)
    sems = (sem_a, sem_b)

    def gather_table(idx_v, sup, table, out):
      copies = [
          pltpu.async_copy(table.at[sup.at[c]], blks[c % 2], sems[c % 2])
          for c in range(2)
      ]

      for c in range(NCH):
        copies[c].wait()
        blk = blks[c % 2]
        cb = c * CH

        def body(g, _, blk=blk, cb=cb):
          gb = g * LANES
          vec = idx_v[pl.ds(cb + gb, LANES)]
          for j in range(LANES):
            half = vec[j] & 1
            t = gb + j
            off = half * EMBED_DIM
            for k in range(VPR):
              row[t, pl.ds(k * LANES, LANES)] = (
                  blk[t, pl.ds(off + k * LANES, LANES)])
          return _

        lax.fori_loop(0, CH // LANES, body, 0)
        pltpu.sync_copy(row, out.at[pl.ds(base + cb, CH)])
        if c + 2 < NCH:
          copies.append(
              pltpu.async_copy(table.at[sup.at[c + 2]], blks[c % 2],
                               sems[c % 2]))

    gather_table(hidx_v, hsup, ent_hbm, out_h)
    gather_table(ridx_v, rsup, rel_hbm, out_r)

  return lookup


_lookup = _make_sc_lookup()


@jax.jit
def kernel(heads, relations, entity_table, relation_table):
  heads_r = heads.astype(jnp.int32).reshape(NW, BPW)
  rels_r = relations.astype(jnp.int32).reshape(NW, BPW)
  ent2 = entity_table.reshape(NUM_ENTITIES // 2, WIDE)
  rel2 = relation_table.reshape(NUM_RELATIONS // 2, WIDE)
  return _lookup(heads_r, rels_r, ent2, rel2)


# speedup vs baseline: 1.6964x; 1.6964x over previous
import functools

import jax
import jax.numpy as jnp
from jax import lax
from jax.experimental import pallas as pl
from jax.experimental.pallas import tpu as pltpu
from jax.experimental.pallas import tpu_sc as plsc

NUM_ENTITIES = 1000000
NUM_RELATIONS = 1000
EMBED_DIM = 64
BATCH = 16384

NC = 2
NS = 16
NW = NC * NS
BPW = BATCH // NW
LANES = 16
WAVE = 256
NWAVE = BPW // WAVE


def _make_sc_lookup():
  mesh = plsc.VectorSubcoreMesh(core_axis_name="c", subcore_axis_name="s")

  @functools.partial(
      pl.kernel,
      mesh=mesh,
      compiler_params=pltpu.CompilerParams(needs_layout_passes=False),
      out_type=(
          jax.ShapeDtypeStruct((BATCH, EMBED_DIM), jnp.float32),
          jax.ShapeDtypeStruct((BATCH, EMBED_DIM), jnp.float32),
      ),
      scratch_types=[
          pltpu.VMEM((BPW,), jnp.int32),
          pltpu.VMEM((BPW,), jnp.int32),
          pltpu.VMEM((WAVE, EMBED_DIM), jnp.float32),
          pltpu.VMEM((WAVE, EMBED_DIM), jnp.float32),
          pltpu.VMEM_SHARED((NUM_RELATIONS, EMBED_DIM), jnp.float32),
          pltpu.SemaphoreType.DMA,
          pltpu.SemaphoreType.DMA,
      ],
  )
  def lookup(heads_hbm, rels_hbm, ent_hbm, rel_hbm, out_h, out_r,
             hidx_v, ridx_v, hrows, rrows, rtab, hsem, rsem):
    cid = lax.axis_index("c")
    sid = lax.axis_index("s")
    wid = sid * NC + cid
    base = wid * BPW
    pltpu.sync_copy(heads_hbm.at[wid], hidx_v)
    pltpu.sync_copy(rels_hbm.at[wid], ridx_v)

    def fire(idx_v, table, rows, sem, wbase):
      def body(g, _):
        vec = idx_v[pl.ds(wbase + g * LANES, LANES)]
        for j in range(LANES):
          pltpu.async_copy(table.at[pl.ds(vec[j], 1)],
                           rows.at[pl.ds(g * LANES + j, 1)], sem)
        return _
      lax.fori_loop(0, WAVE // LANES, body, 0)

    def drain(table, rows, sem):
      def body(i, _):
        pltpu.make_async_copy(table.at[pl.ds(0, 1)],
                              rows.at[pl.ds(i, 1)], sem).wait()
        return _
      lax.fori_loop(0, WAVE, body, 0)

    fire(hidx_v, ent_hbm, hrows, hsem, 0)

    @pl.when(sid == 0)
    def _():
      pltpu.sync_copy(rel_hbm, rtab)

    plsc.subcore_barrier()

    fire(ridx_v, rtab, rrows, rsem, 0)
    drain(ent_hbm, hrows, hsem)
    pltpu.sync_copy(hrows, out_h.at[pl.ds(base, WAVE)])
    fire(hidx_v, ent_hbm, hrows, hsem, WAVE)
    drain(rtab, rrows, rsem)
    pltpu.sync_copy(rrows, out_r.at[pl.ds(base, WAVE)])
    fire(ridx_v, rtab, rrows, rsem, WAVE)
    drain(ent_hbm, hrows, hsem)
    pltpu.sync_copy(hrows, out_h.at[pl.ds(base + WAVE, WAVE)])
    drain(rtab, rrows, rsem)
    pltpu.sync_copy(rrows, out_r.at[pl.ds(base + WAVE, WAVE)])

  return lookup


_lookup = _make_sc_lookup()


@jax.jit
def kernel(heads, relations, entity_table, relation_table):
  heads_r = heads.astype(jnp.int32).reshape(NW, BPW)
  rels_r = relations.astype(jnp.int32).reshape(NW, BPW)
  return _lookup(heads_r, rels_r, entity_table, relation_table)
